# baseline (speedup 1.0000x reference)
import jax
import jax.numpy as jnp
from jax import lax
from jax.experimental import pallas as pl
from jax.experimental.pallas import tpu as pltpu
from jax.experimental.pallas import tpu_sc as plsc

N = 10000
E = 320000
NFEAT = 128
NHID = 128
NHEADS = 4
DH = 32
ALPHA = 0.2

NC = 2
NS = 16
NW = NC * NS
EPW = E // NW
CH = 80
NCHUNK = EPW // CH
NP = 10240
RPT = NP // NS
WEXT = NHID + 16


def _proj_body(x_ref, wt_ref, b_ref, a_ref, ab_ref, whx_ref, sdst_ref):
    wh = jnp.dot(x_ref[...], wt_ref[...], preferred_element_type=jnp.float32)
    wh = wh + b_ref[...]
    s = jnp.dot(wh, a_ref[...], preferred_element_type=jnp.float32) + ab_ref[...]
    z = jnp.zeros((s.shape[0], 12), jnp.float32)
    whx_ref[...] = jnp.concatenate([wh, s[:, 0:4], z], axis=1)
    sdst_ref[...] = jnp.concatenate([s[:, 4:8], z], axis=1)


def _projection(x, wt, bvec, amat, abvec):
    blk = 400
    grid = (N // blk,)
    return pl.pallas_call(
        _proj_body,
        grid=grid,
        in_specs=[
            pl.BlockSpec((blk, NFEAT), lambda i: (i, 0)),
            pl.BlockSpec((NFEAT, NHID), lambda i: (0, 0)),
            pl.BlockSpec((1, NHID), lambda i: (0, 0)),
            pl.BlockSpec((NHID, 8), lambda i: (0, 0)),
            pl.BlockSpec((1, 8), lambda i: (0, 0)),
        ],
        out_specs=[
            pl.BlockSpec((blk, WEXT), lambda i: (i, 0)),
            pl.BlockSpec((blk, 16), lambda i: (i, 0)),
        ],
        out_shape=[
            jax.ShapeDtypeStruct((N, WEXT), jnp.float32),
            jax.ShapeDtypeStruct((N, 16), jnp.float32),
        ],
    )(x, wt, bvec, amat, abvec)


def _lane_bcast(pv, h):
    return lax.gather(
        pv,
        jnp.full((16, 1), h, jnp.int32),
        lax.GatherDimensionNumbers(
            offset_dims=(), collapsed_slice_dims=(0,), start_index_map=(0,)
        ),
        slice_sizes=(1,),
        mode=lax.GatherScatterMode.PROMISE_IN_BOUNDS,
    )


def _edge_body(ei_hbm, whx_hbm, sdst_hbm, accp_hbm,
               isb0, isb1, isb2, isb3, idb0, idb1, idb2, idb3,
               sd0, sd1, rows0, rows1,
               acc_sh,
               isem0, isem1, isem2, isem3, gsem0, gsem1, ssem0, ssem1):
    c = lax.axis_index("c")
    s = lax.axis_index("s")
    wid = s * NC + c
    isb = (isb0, isb1, isb2, isb3)
    idb = (idb0, idb1, idb2, idb3)
    sd = (sd0, sd1)
    rows = (rows0, rows1)
    isem = (isem0, isem1, isem2, isem3)
    gsem = (gsem0, gsem1)
    ssem = (ssem0, ssem1)

    def zr(i, carry):
        for jj in range(WEXT // 16):
            rows0[i, pl.ds(jj * 16, 16)] = jnp.zeros((16,), jnp.float32)
        return carry

    lax.fori_loop(0, CH, zr, 0)
    for t in range(RPT // CH):
        pltpu.sync_copy(rows0, acc_sh.at[pl.ds(s * RPT + t * CH, CH)])
    plsc.subcore_barrier()

    def issue_idx(ci, ib):
        base = wid * EPW + ci * CH
        pltpu.async_copy(ei_hbm.at[0].at[pl.ds(base, CH)], isb[ib], isem[ib])
        pltpu.async_copy(ei_hbm.at[1].at[pl.ds(base, CH)], idb[ib], isem[ib])

    def wait_idx(ib):
        pltpu.make_async_copy(ei_hbm.at[0].at[pl.ds(0, CH)], isb[ib], isem[ib]).wait()
        pltpu.make_async_copy(ei_hbm.at[1].at[pl.ds(0, CH)], idb[ib], isem[ib]).wait()

    def issue_gathers(db, ib):
        pltpu.async_copy(whx_hbm.at[isb[ib]], rows[db], gsem[db])
        pltpu.async_copy(sdst_hbm.at[idb[ib]], sd[db], gsem[db])

    def wait_gathers(db, ib):
        pltpu.make_async_copy(whx_hbm.at[isb[ib]], rows[db], gsem[db]).wait()
        pltpu.make_async_copy(sdst_hbm.at[idb[ib]], sd[db], gsem[db]).wait()

    def compute(db):
        @plsc.parallel_loop(0, CH, unroll=2)
        def body(e):
            js_p = pl.ds(NHID, 16)
            sv = rows[db][e, js_p] + sd[db][e, :]
            ev = jnp.maximum(sv, ALPHA * sv)
            pv = jnp.exp(ev)
            rows[db][e, js_p] = pv
            for j in range(8):
                w = _lane_bcast(pv, j // 2)
                js = pl.ds(j * 16, 16)
                rows[db][e, js] = rows[db][e, js] * w

    def wait_scatter(db, ib):
        pltpu.make_async_copy(rows[db], acc_sh.at[idb[ib]], ssem[db]).wait()

    def process(ci, k, ws, pg, pi):
        db = k & 1
        ib_next = (k + 1) % 4
        ib_nn = (k + 2) % 4
        wait_gathers(db, k)
        if ws:
            wait_scatter(1 - db, (k + 3) % 4)
        if pg:
            wait_idx(ib_next)
            issue_gathers(1 - db, ib_next)
        compute(db)
        pltpu.async_copy(rows[db], acc_sh.at[idb[k]], ssem[db], add=True)
        if pi:
            issue_idx(ci + 2, ib_nn)

    issue_idx(0, 0)
    issue_idx(1, 1)
    wait_idx(0)
    issue_gathers(0, 0)
    process(0, 0, False, True, True)
    process(1, 1, True, True, True)
    process(2, 2, True, True, True)
    process(3, 3, True, True, True)

    def outer(i, carry):
        c0 = 4 * i
        for k in range(4):
            process(c0 + k, k, True, True, True)
        return carry

    lax.fori_loop(1, NCHUNK // 4, outer, 0)
    process(NCHUNK - 1, 0, True, False, False)
    wait_scatter(0, 0)
    plsc.subcore_barrier()

    sl = pl.ds(s * RPT, RPT)
    pltpu.sync_copy(acc_sh.at[sl], accp_hbm.at[c].at[sl])


def _edge_pass(ei, whx, sdst):
    mesh = plsc.VectorSubcoreMesh(
        core_axis_name="c", subcore_axis_name="s", num_cores=NC, num_subcores=NS
    )
    f = pl.kernel(
        _edge_body,
        out_type=jax.ShapeDtypeStruct((NC, NP, WEXT), jnp.float32),
        mesh=mesh,
        compiler_params=pltpu.CompilerParams(use_tc_tiling_on_sc=False),
        scratch_types=(
            [pltpu.VMEM((CH,), jnp.int32)] * 8
            + [pltpu.VMEM((CH, 16), jnp.float32)] * 2
            + [pltpu.VMEM((CH, WEXT), jnp.float32)] * 2
            + [pltpu.VMEM_SHARED((NP, WEXT), jnp.float32)]
            + [pltpu.SemaphoreType.DMA] * 8
        ),
    )
    return f(ei, whx, sdst)


def _combine_body(accp_ref, o_ref):
    num = accp_ref[0][:, 0:NHID] + accp_ref[1][:, 0:NHID]
    d = accp_ref[0][:, NHID:NHID + 4] + accp_ref[1][:, NHID:NHID + 4]
    d = jnp.where(d > 0.0, d, 1.0)
    dfull = jnp.concatenate(
        [jnp.broadcast_to(d[:, h:h + 1], (num.shape[0], DH)) for h in range(NHEADS)],
        axis=1,
    )
    o_ref[...] = num / dfull


def _combine(accp):
    blk = 400
    grid = (N // blk,)
    return pl.pallas_call(
        _combine_body,
        grid=grid,
        in_specs=[pl.BlockSpec((NC, blk, WEXT), lambda i: (0, i, 0))],
        out_specs=pl.BlockSpec((blk, NHID), lambda i: (i, 0)),
        out_shape=jax.ShapeDtypeStruct((N, NHID), jnp.float32),
    )(accp)


@jax.jit
def kernel(x, edge_index, W_w, W_b, a_w, a_b):
    ei = edge_index.astype(jnp.int32)

    wt = jnp.transpose(W_w, (2, 0, 1)).reshape(NFEAT, NHID)
    bvec = W_b.reshape(1, NHID)
    amat = jnp.zeros((NHID, 8), jnp.float32)
    for h in range(NHEADS):
        amat = amat.at[h * DH:(h + 1) * DH, h].set(a_w[h, 0, :DH])
        amat = amat.at[h * DH:(h + 1) * DH, 4 + h].set(a_w[h, 0, DH:])
    abvec = jnp.concatenate([jnp.zeros((4,), jnp.float32), a_b[:, 0]]).reshape(1, 8)

    whx, sdst = _projection(x, wt, bvec, amat, abvec)
    accp = _edge_pass(ei, whx, sdst)
    return _combine(accp)


# speedup vs baseline: 1.0644x; 1.0644x over previous
import jax
import jax.numpy as jnp
from jax import lax
from jax.experimental import pallas as pl
from jax.experimental.pallas import tpu as pltpu
from jax.experimental.pallas import tpu_sc as plsc

N = 10000
E = 320000
NFEAT = 128
NHID = 128
NHEADS = 4
DH = 32
ALPHA = 0.2

NC = 2
NS = 16
NW = NC * NS
EPW = E // NW
CH = 80
NCHUNK = EPW // CH
NP = 10240
RPT = NP // NS
WEXT = NHID + 16


def _proj_body(x_ref, wt_ref, b_ref, a_ref, ab_ref, wh_ref, ssrc_ref, sdst_ref):
    wh = jnp.dot(x_ref[...], wt_ref[...], preferred_element_type=jnp.float32)
    wh = wh + b_ref[...]
    s = jnp.dot(wh, a_ref[...], preferred_element_type=jnp.float32) + ab_ref[...]
    z = jnp.zeros((s.shape[0], 12), jnp.float32)
    wh_ref[...] = wh
    ssrc_ref[...] = jnp.concatenate([s[:, 0:4], z], axis=1)
    sdst_ref[...] = jnp.concatenate([s[:, 4:8], z], axis=1)


def _projection(x, wt, bvec, amat, abvec):
    blk = 400
    grid = (N // blk,)
    return pl.pallas_call(
        _proj_body,
        grid=grid,
        in_specs=[
            pl.BlockSpec((blk, NFEAT), lambda i: (i, 0)),
            pl.BlockSpec((NFEAT, NHID), lambda i: (0, 0)),
            pl.BlockSpec((1, NHID), lambda i: (0, 0)),
            pl.BlockSpec((NHID, 8), lambda i: (0, 0)),
            pl.BlockSpec((1, 8), lambda i: (0, 0)),
        ],
        out_specs=[
            pl.BlockSpec((blk, NHID), lambda i: (i, 0)),
            pl.BlockSpec((blk, 16), lambda i: (i, 0)),
            pl.BlockSpec((blk, 16), lambda i: (i, 0)),
        ],
        out_shape=[
            jax.ShapeDtypeStruct((N, NHID), jnp.float32),
            jax.ShapeDtypeStruct((N, 16), jnp.float32),
            jax.ShapeDtypeStruct((N, 16), jnp.float32),
        ],
    )(x, wt, bvec, amat, abvec)


def _lane_bcast(pv, h):
    return lax.gather(
        pv,
        jnp.full((16, 1), h, jnp.int32),
        lax.GatherDimensionNumbers(
            offset_dims=(), collapsed_slice_dims=(0,), start_index_map=(0,)
        ),
        slice_sizes=(1,),
        mode=lax.GatherScatterMode.PROMISE_IN_BOUNDS,
    )


def _edge_body(ei_hbm, wh_hbm, ssrc_hbm, sdst_hbm, outp_hbm, denp_hbm,
               isb0, isb1, isb2, isb3, idb0, idb1, idb2, idb3,
               ss0, ss1, sd0, sd1, rows0, rows1, p,
               out_sh, den_sh,
               isem0, isem1, isem2, isem3, gsem0, gsem1, ssem0, ssem1):
    c = lax.axis_index("c")
    s = lax.axis_index("s")
    wid = s * NC + c
    isb = (isb0, isb1, isb2, isb3)
    idb = (idb0, idb1, idb2, idb3)
    sd = (sd0, sd1)
    rows = (rows0, rows1)
    ss = (ss0, ss1)
    isem = (isem0, isem1, isem2, isem3)
    gsem = (gsem0, gsem1)
    ssem = (ssem0, ssem1)

    def zr(i, carry):
        for jj in range(NHID // 16):
            rows0[i, pl.ds(jj * 16, 16)] = jnp.zeros((16,), jnp.float32)
        p[i, :] = jnp.zeros((16,), jnp.float32)
        return carry

    lax.fori_loop(0, CH, zr, 0)
    for t in range(RPT // CH):
        pltpu.sync_copy(rows0, out_sh.at[pl.ds(s * RPT + t * CH, CH)])
        pltpu.sync_copy(p, den_sh.at[pl.ds(s * RPT + t * CH, CH)])
    plsc.subcore_barrier()

    def issue_idx(ci, ib):
        base = wid * EPW + ci * CH
        pltpu.async_copy(ei_hbm.at[0].at[pl.ds(base, CH)], isb[ib], isem[ib])
        pltpu.async_copy(ei_hbm.at[1].at[pl.ds(base, CH)], idb[ib], isem[ib])

    def wait_idx(ib):
        pltpu.make_async_copy(ei_hbm.at[0].at[pl.ds(0, CH)], isb[ib], isem[ib]).wait()
        pltpu.make_async_copy(ei_hbm.at[1].at[pl.ds(0, CH)], idb[ib], isem[ib]).wait()

    def issue_gathers(db, ib):
        pltpu.async_copy(wh_hbm.at[isb[ib]], rows[db], gsem[db])
        pltpu.async_copy(ssrc_hbm.at[isb[ib]], ss[db], gsem[db])
        pltpu.async_copy(sdst_hbm.at[idb[ib]], sd[db], gsem[db])

    def wait_gathers(db, ib):
        pltpu.make_async_copy(wh_hbm.at[isb[ib]], rows[db], gsem[db]).wait()
        pltpu.make_async_copy(ssrc_hbm.at[isb[ib]], ss[db], gsem[db]).wait()
        pltpu.make_async_copy(sdst_hbm.at[idb[ib]], sd[db], gsem[db]).wait()

    def compute(db):
        @plsc.parallel_loop(0, CH, unroll=2)
        def body(e):
            sv = ss[db][e, :] + sd[db][e, :]
            ev = jnp.maximum(sv, ALPHA * sv)
            pv = jnp.exp(ev)
            p[e, :] = pv
            for j in range(8):
                w = _lane_bcast(pv, j // 2)
                js = pl.ds(j * 16, 16)
                rows[db][e, js] = rows[db][e, js] * w

    def wait_scatter(db, ib):
        pltpu.make_async_copy(rows[db], out_sh.at[idb[ib]], ssem[db]).wait()

    def process(ci, k, ws, pg, pi):
        db = k & 1
        ib_next = (k + 1) % 4
        ib_nn = (k + 2) % 4
        wait_gathers(db, k)
        if ws:
            wait_scatter(1 - db, (k + 3) % 4)
        if pg:
            wait_idx(ib_next)
            issue_gathers(1 - db, ib_next)
        compute(db)
        pltpu.sync_copy(p, den_sh.at[idb[k]], add=True)
        pltpu.async_copy(rows[db], out_sh.at[idb[k]], ssem[db], add=True)
        if pi:
            issue_idx(ci + 2, ib_nn)

    issue_idx(0, 0)
    issue_idx(1, 1)
    wait_idx(0)
    issue_gathers(0, 0)
    process(0, 0, False, True, True)
    process(1, 1, True, True, True)
    process(2, 2, True, True, True)
    process(3, 3, True, True, True)

    def outer(i, carry):
        c0 = 4 * i
        for k in range(4):
            process(c0 + k, k, True, True, True)
        return carry

    lax.fori_loop(1, NCHUNK // 4, outer, 0)
    process(NCHUNK - 1, 0, True, False, False)
    wait_scatter(0, 0)
    plsc.subcore_barrier()

    sl = pl.ds(s * RPT, RPT)
    pltpu.sync_copy(out_sh.at[sl], outp_hbm.at[c].at[sl])
    pltpu.sync_copy(den_sh.at[sl], denp_hbm.at[c].at[sl])


def _edge_pass(ei, wh, ssrc, sdst):
    mesh = plsc.VectorSubcoreMesh(
        core_axis_name="c", subcore_axis_name="s", num_cores=NC, num_subcores=NS
    )
    f = pl.kernel(
        _edge_body,
        out_type=[
            jax.ShapeDtypeStruct((NC, NP, NHID), jnp.float32),
            jax.ShapeDtypeStruct((NC, NP, 16), jnp.float32),
        ],
        mesh=mesh,
        compiler_params=pltpu.CompilerParams(use_tc_tiling_on_sc=False),
        scratch_types=(
            [pltpu.VMEM((CH,), jnp.int32)] * 8
            + [pltpu.VMEM((CH, 16), jnp.float32)] * 4
            + [pltpu.VMEM((CH, NHID), jnp.float32)] * 2
            + [pltpu.VMEM((CH, 16), jnp.float32)]
            + [
                pltpu.VMEM_SHARED((NP, NHID), jnp.float32),
                pltpu.VMEM_SHARED((NP, 16), jnp.float32),
            ]
            + [pltpu.SemaphoreType.DMA] * 8
        ),
    )
    return f(ei, wh, ssrc, sdst)


def _combine_body(outp_ref, denp_ref, o_ref):
    num = outp_ref[0] + outp_ref[1]
    d = denp_ref[0][:, 0:4] + denp_ref[1][:, 0:4]
    d = jnp.where(d > 0.0, d, 1.0)
    dfull = jnp.concatenate(
        [jnp.broadcast_to(d[:, h:h + 1], (num.shape[0], DH)) for h in range(NHEADS)],
        axis=1,
    )
    o_ref[...] = num / dfull


def _combine(outp, denp):
    blk = 400
    grid = (N // blk,)
    return pl.pallas_call(
        _combine_body,
        grid=grid,
        in_specs=[
            pl.BlockSpec((NC, blk, NHID), lambda i: (0, i, 0)),
            pl.BlockSpec((NC, blk, 16), lambda i: (0, i, 0)),
        ],
        out_specs=pl.BlockSpec((blk, NHID), lambda i: (i, 0)),
        out_shape=jax.ShapeDtypeStruct((N, NHID), jnp.float32),
    )(outp, denp)


@jax.jit
def kernel(x, edge_index, W_w, W_b, a_w, a_b):
    ei = edge_index.astype(jnp.int32)

    wt = jnp.transpose(W_w, (2, 0, 1)).reshape(NFEAT, NHID)
    bvec = W_b.reshape(1, NHID)
    amat = jnp.zeros((NHID, 8), jnp.float32)
    for h in range(NHEADS):
        amat = amat.at[h * DH:(h + 1) * DH, h].set(a_w[h, 0, :DH])
        amat = amat.at[h * DH:(h + 1) * DH, 4 + h].set(a_w[h, 0, DH:])
    abvec = jnp.concatenate([jnp.zeros((4,), jnp.float32), a_b[:, 0]]).reshape(1, 8)

    wh, ssrc, sdst = _projection(x, wt, bvec, amat, abvec)
    outp, denp = _edge_pass(ei, wh, ssrc, sdst)
    return _combine(outp, denp)
